# baseline (speedup 1.0000x reference)
import functools

import jax
import jax.numpy as jnp
from jax import lax
from jax.experimental import pallas as pl
from jax.experimental.pallas import tpu as pltpu
from jax.experimental.pallas import tpu_sc as plsc

_B = 64
_T = 200
_E = 128
_H = 128
_NW = 32
_CH = 80
_RPW = 5


def _sc_gather_permute(tok2d, sidx2d, emb, n):
    mesh = plsc.VectorSubcoreMesh(core_axis_name="c", subcore_axis_name="s")

    @functools.partial(
        pl.kernel,
        mesh=mesh,
        out_type=jax.ShapeDtypeStruct((n, _E), jnp.float32),
        scratch_types=[
            pltpu.VMEM((_RPW, _CH), jnp.int32),
            pltpu.VMEM((_RPW, _CH), jnp.int32),
            pltpu.VMEM((_RPW * _CH, _E), jnp.float32),
            pltpu.SemaphoreType.DMA,
            pltpu.SemaphoreType.DMA,
        ],
    )
    def gather_kernel(tok_hbm, sidx_hbm, emb_hbm, out_hbm, tok_v, dst_v, rows_v,
                      sem, sem2):
        wid = lax.axis_index("s") * 2 + lax.axis_index("c")
        pltpu.sync_copy(tok_hbm.at[wid], tok_v)
        pltpu.sync_copy(sidx_hbm.at[wid], dst_v)
        vT = jnp.full((16,), _T, jnp.int32)
        vB = jnp.full((16,), _B, jnp.int32)
        for i in range(_RPW):
            for j in range(_CH // 16):
                s = dst_v[i, pl.ds(j * 16, 16)]
                dst_v[i, pl.ds(j * 16, 16)] = (
                    lax.rem(s, vT) * vB + lax.div(s, vT)
                )
        half = _RPW // 2
        ga = [
            pltpu.async_copy(
                emb_hbm.at[tok_v.at[i]], rows_v.at[pl.ds(i * _CH, _CH)], sem
            )
            for i in range(half)
        ]
        for g in ga:
            g.wait()
        gb = [
            pltpu.async_copy(
                emb_hbm.at[tok_v.at[i]], rows_v.at[pl.ds(i * _CH, _CH)], sem
            )
            for i in range(half, _RPW)
        ]
        pa = [
            pltpu.async_copy(
                rows_v.at[pl.ds(i * _CH, _CH)], out_hbm.at[dst_v.at[i]], sem2
            )
            for i in range(half)
        ]
        for g in gb:
            g.wait()
        pb = [
            pltpu.async_copy(
                rows_v.at[pl.ds(i * _CH, _CH)], out_hbm.at[dst_v.at[i]], sem2
            )
            for i in range(half, _RPW)
        ]
        for p in pa + pb:
            p.wait()

    return gather_kernel(tok2d, sidx2d, emb)


def _tc_gru(xt, wc, bc, wih_f, bih_f, whh_f, bhh_f,
            wih_b, bih_b, whh_b, bhh_b, w2l, b2l):
    n = xt.shape[0]
    l_out = w2l.shape[0]

    def body(xt_ref, wc_ref, bc_ref, wihf_ref, bihf_ref, whhf_ref, bhhf_ref,
             wihb_ref, bihb_ref, whhb_ref, bhhb_ref, w2l_ref, b2l_ref,
             o_ref, x2_scr, gf_scr, gb_scr):
        dn = (((1,), (1,)), ((), ()))
        x2_scr[...] = (
            lax.dot_general(xt_ref[...].astype(jnp.bfloat16),
                            wc_ref[...].astype(jnp.bfloat16), dn,
                            preferred_element_type=jnp.float32)
            + bc_ref[...]
        )
        x2b = x2_scr[...].astype(jnp.bfloat16)
        gf_scr[...] = (
            lax.dot_general(x2b, wihf_ref[...].astype(jnp.bfloat16), dn,
                            preferred_element_type=jnp.float32)
            + bihf_ref[...]
        )
        gb_scr[...] = (
            lax.dot_general(x2b, wihb_ref[...].astype(jnp.bfloat16), dn,
                            preferred_element_type=jnp.float32)
            + bihb_ref[...]
        )
        whhf_m = whhf_ref[...].astype(jnp.bfloat16)
        whhb_m = whhb_ref[...].astype(jnp.bfloat16)
        bhhf_n = bhhf_ref[:, 2 * _H:]
        bhhb_n = bhhb_ref[:, 2 * _H:]

        def cell(gi, h, hbf, whh_m, bhhn_v):
            gh_rz = lax.dot_general(hbf, whh_m[:2 * _H, :], dn,
                                    preferred_element_type=jnp.float32)
            gh_n = lax.dot_general(hbf, whh_m[2 * _H:, :], dn,
                                   preferred_element_type=jnp.float32)
            r = jax.nn.sigmoid(gi[:, :_H] + gh_rz[:, :_H])
            z = jax.nn.sigmoid(gi[:, _H:2 * _H] + gh_rz[:, _H:])
            nn = jnp.tanh(gi[:, 2 * _H:] + r * (gh_n + bhhn_v))
            h2 = nn + z * (h - nn)
            return h2, h2.astype(jnp.bfloat16)

        def step(t, carry):
            hf, hb, hfb, hbb, mf, mb = carry
            af = gf_scr[pl.ds(t * _B, _B), :]
            ab = gb_scr[pl.ds((_T - 1 - t) * _B, _B), :]
            hf2, hfb2 = cell(af, hf, hfb, whhf_m, bhhf_n)
            hb2, hbb2 = cell(ab, hb, hbb, whhb_m, bhhb_n)
            return (hf2, hb2, hfb2, hbb2,
                    jnp.maximum(mf, hf2), jnp.maximum(mb, hb2))

        h0 = jnp.zeros((_B, _H), jnp.float32)
        h0b = jnp.zeros((_B, _H), jnp.bfloat16)
        m0 = jnp.full((_B, _H), -jnp.inf, jnp.float32)
        _, _, _, _, mf, mb = lax.fori_loop(
            0, _T, step, (h0, h0, h0b, h0b, m0, m0), unroll=8)
        pooled = jnp.concatenate([mf, mb], axis=1)
        o_ref[...] = (
            lax.dot_general(pooled, w2l_ref[...], dn,
                            preferred_element_type=jnp.float32)
            + b2l_ref[...]
        )

    return pl.pallas_call(
        body,
        out_shape=jax.ShapeDtypeStruct((_B, l_out), jnp.float32),
        scratch_shapes=[
            pltpu.VMEM((n, _E), jnp.float32),
            pltpu.VMEM((n, 3 * _H), jnp.float32),
            pltpu.VMEM((n, 3 * _H), jnp.float32),
        ],
    )(xt, wc, bc, wih_f, bih_f, whh_f, bhh_f,
      wih_b, bih_b, whh_b, bhh_b, w2l, b2l)


def kernel(tokens, scatter_idx, emb, W_c_w, W_c_b,
           W_ih_f, W_hh_f, b_ih_f, b_hh_f,
           W_ih_b, W_hh_b, b_ih_b, b_hh_b,
           h2l_w, h2l_b):
    n = tokens.shape[0]
    tok2d = tokens.astype(jnp.int32).reshape(_NW, _RPW, _CH)
    sidx2d = scatter_idx.astype(jnp.int32).reshape(_NW, _RPW, _CH)
    xt = _sc_gather_permute(tok2d, sidx2d, emb, n)

    z_h = jnp.zeros((_H,), jnp.float32)
    bf = b_ih_f + jnp.concatenate([b_hh_f[:2 * _H], z_h])
    bb = b_ih_b + jnp.concatenate([b_hh_b[:2 * _H], z_h])
    return _tc_gru(xt, W_c_w, W_c_b.reshape(1, _E),
                   W_ih_f, bf.reshape(1, 3 * _H),
                   W_hh_f, b_hh_f.reshape(1, 3 * _H),
                   W_ih_b, bb.reshape(1, 3 * _H),
                   W_hh_b, b_hh_b.reshape(1, 3 * _H),
                   h2l_w, h2l_b.reshape(1, h2l_b.shape[0]))


# speedup vs baseline: 1.0129x; 1.0129x over previous
import functools

import jax
import jax.numpy as jnp
from jax import lax
from jax.experimental import pallas as pl
from jax.experimental.pallas import tpu as pltpu
from jax.experimental.pallas import tpu_sc as plsc

_B = 64
_T = 200
_E = 128
_H = 128
_NW = 32
_CH = 80
_RPW = 5


def _sc_gather_permute(tok2d, sidx2d, emb, n):
    mesh = plsc.VectorSubcoreMesh(core_axis_name="c", subcore_axis_name="s")

    @functools.partial(
        pl.kernel,
        mesh=mesh,
        out_type=jax.ShapeDtypeStruct((n, _E), jnp.float32),
        scratch_types=[
            pltpu.VMEM((_RPW, _CH), jnp.int32),
            pltpu.VMEM((_RPW, _CH), jnp.int32),
            pltpu.VMEM((_RPW * _CH, _E), jnp.float32),
            pltpu.SemaphoreType.DMA,
            pltpu.SemaphoreType.DMA,
        ],
    )
    def gather_kernel(tok_hbm, sidx_hbm, emb_hbm, out_hbm, tok_v, dst_v, rows_v,
                      sem, sem2):
        wid = lax.axis_index("s") * 2 + lax.axis_index("c")
        pltpu.sync_copy(tok_hbm.at[wid], tok_v)
        pltpu.sync_copy(sidx_hbm.at[wid], dst_v)
        vT = jnp.full((16,), _T, jnp.int32)
        vB = jnp.full((16,), _B, jnp.int32)
        for i in range(_RPW):
            for j in range(_CH // 16):
                s = dst_v[i, pl.ds(j * 16, 16)]
                dst_v[i, pl.ds(j * 16, 16)] = (
                    lax.rem(s, vT) * vB + lax.div(s, vT)
                )
        half = _RPW // 2
        ga = [
            pltpu.async_copy(
                emb_hbm.at[tok_v.at[i]], rows_v.at[pl.ds(i * _CH, _CH)], sem
            )
            for i in range(half)
        ]
        for g in ga:
            g.wait()
        gb = [
            pltpu.async_copy(
                emb_hbm.at[tok_v.at[i]], rows_v.at[pl.ds(i * _CH, _CH)], sem
            )
            for i in range(half, _RPW)
        ]
        pa = [
            pltpu.async_copy(
                rows_v.at[pl.ds(i * _CH, _CH)], out_hbm.at[dst_v.at[i]], sem2
            )
            for i in range(half)
        ]
        for g in gb:
            g.wait()
        pb = [
            pltpu.async_copy(
                rows_v.at[pl.ds(i * _CH, _CH)], out_hbm.at[dst_v.at[i]], sem2
            )
            for i in range(half, _RPW)
        ]
        for p in pa + pb:
            p.wait()

    return gather_kernel(tok2d, sidx2d, emb)


def _tc_gru(xt, wc, bc, wih_f, bih_f, whh_f, bhh_f,
            wih_b, bih_b, whh_b, bhh_b, w2l, b2l):
    n = xt.shape[0]
    l_out = w2l.shape[0]

    def body(xt_ref, wc_ref, bc_ref, wihf_ref, bihf_ref, whhf_ref, bhhf_ref,
             wihb_ref, bihb_ref, whhb_ref, bhhb_ref, w2l_ref, b2l_ref,
             o_ref, x2_scr, gf_scr, gb_scr):
        dn = (((1,), (1,)), ((), ()))
        x2_scr[...] = (
            lax.dot_general(xt_ref[...].astype(jnp.bfloat16),
                            wc_ref[...].astype(jnp.bfloat16), dn,
                            preferred_element_type=jnp.float32)
            + bc_ref[...]
        )
        x2b = x2_scr[...].astype(jnp.bfloat16)
        gf_scr[...] = (
            lax.dot_general(x2b, wihf_ref[...].astype(jnp.bfloat16), dn,
                            preferred_element_type=jnp.float32)
            + bihf_ref[...]
        )
        gb_scr[...] = (
            lax.dot_general(x2b, wihb_ref[...].astype(jnp.bfloat16), dn,
                            preferred_element_type=jnp.float32)
            + bihb_ref[...]
        )
        whhf_m = whhf_ref[...].astype(jnp.bfloat16)
        whhb_m = whhb_ref[...].astype(jnp.bfloat16)
        bhhf_n = bhhf_ref[:, 2 * _H:]
        bhhb_n = bhhb_ref[:, 2 * _H:]

        def cell(gi, h, hbf, whh_m, bhhn_v):
            gh_rz = lax.dot_general(hbf, whh_m[:2 * _H, :], dn,
                                    preferred_element_type=jnp.float32)
            gh_n = lax.dot_general(hbf, whh_m[2 * _H:, :], dn,
                                   preferred_element_type=jnp.float32)
            r = jax.nn.sigmoid(gi[:, :_H] + gh_rz[:, :_H])
            z = jax.nn.sigmoid(gi[:, _H:2 * _H] + gh_rz[:, _H:])
            nn = jnp.tanh(gi[:, 2 * _H:] + r * (gh_n + bhhn_v))
            h2 = nn + z * (h - nn)
            return h2, h2.astype(jnp.bfloat16)

        def step(t, carry):
            hf, hb, hfb, hbb, mf, mb = carry
            af = gf_scr[pl.ds(t * _B, _B), :]
            ab = gb_scr[pl.ds((_T - 1 - t) * _B, _B), :]
            hf2, hfb2 = cell(af, hf, hfb, whhf_m, bhhf_n)
            hb2, hbb2 = cell(ab, hb, hbb, whhb_m, bhhb_n)
            return (hf2, hb2, hfb2, hbb2,
                    jnp.maximum(mf, hf2), jnp.maximum(mb, hb2))

        h0 = jnp.zeros((_B, _H), jnp.float32)
        h0b = jnp.zeros((_B, _H), jnp.bfloat16)
        m0 = jnp.full((_B, _H), -jnp.inf, jnp.float32)
        _, _, _, _, mf, mb = lax.fori_loop(
            0, _T, step, (h0, h0, h0b, h0b, m0, m0), unroll=16)
        pooled = jnp.concatenate([mf, mb], axis=1)
        o_ref[...] = (
            lax.dot_general(pooled, w2l_ref[...], dn,
                            preferred_element_type=jnp.float32)
            + b2l_ref[...]
        )

    return pl.pallas_call(
        body,
        out_shape=jax.ShapeDtypeStruct((_B, l_out), jnp.float32),
        scratch_shapes=[
            pltpu.VMEM((n, _E), jnp.float32),
            pltpu.VMEM((n, 3 * _H), jnp.float32),
            pltpu.VMEM((n, 3 * _H), jnp.float32),
        ],
    )(xt, wc, bc, wih_f, bih_f, whh_f, bhh_f,
      wih_b, bih_b, whh_b, bhh_b, w2l, b2l)


def kernel(tokens, scatter_idx, emb, W_c_w, W_c_b,
           W_ih_f, W_hh_f, b_ih_f, b_hh_f,
           W_ih_b, W_hh_b, b_ih_b, b_hh_b,
           h2l_w, h2l_b):
    n = tokens.shape[0]
    tok2d = tokens.astype(jnp.int32).reshape(_NW, _RPW, _CH)
    sidx2d = scatter_idx.astype(jnp.int32).reshape(_NW, _RPW, _CH)
    xt = _sc_gather_permute(tok2d, sidx2d, emb, n)

    z_h = jnp.zeros((_H,), jnp.float32)
    bf = b_ih_f + jnp.concatenate([b_hh_f[:2 * _H], z_h])
    bb = b_ih_b + jnp.concatenate([b_hh_b[:2 * _H], z_h])
    return _tc_gru(xt, W_c_w, W_c_b.reshape(1, _E),
                   W_ih_f, bf.reshape(1, 3 * _H),
                   W_hh_f, b_hh_f.reshape(1, 3 * _H),
                   W_ih_b, bb.reshape(1, 3 * _H),
                   W_hh_b, b_hh_b.reshape(1, 3 * _H),
                   h2l_w, h2l_b.reshape(1, h2l_b.shape[0]))


# speedup vs baseline: 1.0146x; 1.0017x over previous
import functools

import jax
import jax.numpy as jnp
from jax import lax
from jax.experimental import pallas as pl
from jax.experimental.pallas import tpu as pltpu
from jax.experimental.pallas import tpu_sc as plsc

_B = 64
_T = 200
_E = 128
_H = 128
_NW = 32
_CH = 80
_RPW = 5


def _sc_gather_permute(tok2d, sidx2d, emb, n):
    mesh = plsc.VectorSubcoreMesh(core_axis_name="c", subcore_axis_name="s")

    @functools.partial(
        pl.kernel,
        mesh=mesh,
        out_type=jax.ShapeDtypeStruct((n, _E), jnp.float32),
        scratch_types=[
            pltpu.VMEM((_RPW, _CH), jnp.int32),
            pltpu.VMEM((_RPW, _CH), jnp.int32),
            pltpu.VMEM((_RPW * _CH, _E), jnp.float32),
            pltpu.SemaphoreType.DMA,
            pltpu.SemaphoreType.DMA,
        ],
    )
    def gather_kernel(tok_hbm, sidx_hbm, emb_hbm, out_hbm, tok_v, dst_v, rows_v,
                      sem, sem2):
        wid = lax.axis_index("s") * 2 + lax.axis_index("c")
        pltpu.sync_copy(tok_hbm.at[wid], tok_v)
        pltpu.sync_copy(sidx_hbm.at[wid], dst_v)
        vT = jnp.full((16,), _T, jnp.int32)
        vB = jnp.full((16,), _B, jnp.int32)
        for i in range(_RPW):
            for j in range(_CH // 16):
                s = dst_v[i, pl.ds(j * 16, 16)]
                dst_v[i, pl.ds(j * 16, 16)] = (
                    lax.rem(s, vT) * vB + lax.div(s, vT)
                )
        half = _RPW // 2
        ga = [
            pltpu.async_copy(
                emb_hbm.at[tok_v.at[i]], rows_v.at[pl.ds(i * _CH, _CH)], sem
            )
            for i in range(half)
        ]
        for g in ga:
            g.wait()
        gb = [
            pltpu.async_copy(
                emb_hbm.at[tok_v.at[i]], rows_v.at[pl.ds(i * _CH, _CH)], sem
            )
            for i in range(half, _RPW)
        ]
        pa = [
            pltpu.async_copy(
                rows_v.at[pl.ds(i * _CH, _CH)], out_hbm.at[dst_v.at[i]], sem2
            )
            for i in range(half)
        ]
        for g in gb:
            g.wait()
        pb = [
            pltpu.async_copy(
                rows_v.at[pl.ds(i * _CH, _CH)], out_hbm.at[dst_v.at[i]], sem2
            )
            for i in range(half, _RPW)
        ]
        for p in pa + pb:
            p.wait()

    return gather_kernel(tok2d, sidx2d, emb)


def _tc_gru(xt, wc, bc, wih_f, bih_f, whh_f, bhh_f,
            wih_b, bih_b, whh_b, bhh_b, w2l, b2l):
    n = xt.shape[0]
    l_out = w2l.shape[0]

    def body(xt_ref, wc_ref, bc_ref, wihf_ref, bihf_ref, whhf_ref, bhhf_ref,
             wihb_ref, bihb_ref, whhb_ref, bhhb_ref, w2l_ref, b2l_ref,
             o_ref, x2_scr, gf_scr, gb_scr):
        dn = (((1,), (1,)), ((), ()))
        x2_scr[...] = (
            lax.dot_general(xt_ref[...].astype(jnp.bfloat16),
                            wc_ref[...].astype(jnp.bfloat16), dn,
                            preferred_element_type=jnp.float32)
            + bc_ref[...]
        )
        x2b = x2_scr[...].astype(jnp.bfloat16)
        gf_scr[...] = (
            lax.dot_general(x2b, wihf_ref[...].astype(jnp.bfloat16), dn,
                            preferred_element_type=jnp.float32)
            + bihf_ref[...]
        )
        gb_scr[...] = (
            lax.dot_general(x2b, wihb_ref[...].astype(jnp.bfloat16), dn,
                            preferred_element_type=jnp.float32)
            + bihb_ref[...]
        )
        whhf_m = whhf_ref[...].astype(jnp.bfloat16)
        whhb_m = whhb_ref[...].astype(jnp.bfloat16)
        bhhf_n = bhhf_ref[:, 2 * _H:]
        bhhb_n = bhhb_ref[:, 2 * _H:]

        def cell(gi, h, hbf, whh_m, bhhn_v):
            gh_rz = lax.dot_general(hbf, whh_m[:2 * _H, :], dn,
                                    preferred_element_type=jnp.float32)
            gh_n = lax.dot_general(hbf, whh_m[2 * _H:, :], dn,
                                   preferred_element_type=jnp.float32)
            r = jax.nn.sigmoid(gi[:, :_H] + gh_rz[:, :_H])
            z = jax.nn.sigmoid(gi[:, _H:2 * _H] + gh_rz[:, _H:])
            nn = jnp.tanh(gi[:, 2 * _H:] + r * (gh_n + bhhn_v))
            h2 = nn + z * (h - nn)
            return h2, h2.astype(jnp.bfloat16)

        def step(t, carry):
            hf, hb, hfb, hbb, mf, mb = carry
            af = gf_scr[pl.ds(t * _B, _B), :]
            ab = gb_scr[pl.ds((_T - 1 - t) * _B, _B), :]
            hf2, hfb2 = cell(af, hf, hfb, whhf_m, bhhf_n)
            hb2, hbb2 = cell(ab, hb, hbb, whhb_m, bhhb_n)
            return (hf2, hb2, hfb2, hbb2,
                    jnp.maximum(mf, hf2), jnp.maximum(mb, hb2))

        h0 = jnp.zeros((_B, _H), jnp.float32)
        h0b = jnp.zeros((_B, _H), jnp.bfloat16)
        m0 = jnp.full((_B, _H), -jnp.inf, jnp.float32)
        _, _, _, _, mf, mb = lax.fori_loop(
            0, _T, step, (h0, h0, h0b, h0b, m0, m0), unroll=20)
        pooled = jnp.concatenate([mf, mb], axis=1)
        o_ref[...] = (
            lax.dot_general(pooled, w2l_ref[...], dn,
                            preferred_element_type=jnp.float32)
            + b2l_ref[...]
        )

    return pl.pallas_call(
        body,
        out_shape=jax.ShapeDtypeStruct((_B, l_out), jnp.float32),
        scratch_shapes=[
            pltpu.VMEM((n, _E), jnp.float32),
            pltpu.VMEM((n, 3 * _H), jnp.float32),
            pltpu.VMEM((n, 3 * _H), jnp.float32),
        ],
    )(xt, wc, bc, wih_f, bih_f, whh_f, bhh_f,
      wih_b, bih_b, whh_b, bhh_b, w2l, b2l)


def kernel(tokens, scatter_idx, emb, W_c_w, W_c_b,
           W_ih_f, W_hh_f, b_ih_f, b_hh_f,
           W_ih_b, W_hh_b, b_ih_b, b_hh_b,
           h2l_w, h2l_b):
    n = tokens.shape[0]
    tok2d = tokens.astype(jnp.int32).reshape(_NW, _RPW, _CH)
    sidx2d = scatter_idx.astype(jnp.int32).reshape(_NW, _RPW, _CH)
    xt = _sc_gather_permute(tok2d, sidx2d, emb, n)

    z_h = jnp.zeros((_H,), jnp.float32)
    bf = b_ih_f + jnp.concatenate([b_hh_f[:2 * _H], z_h])
    bb = b_ih_b + jnp.concatenate([b_hh_b[:2 * _H], z_h])
    return _tc_gru(xt, W_c_w, W_c_b.reshape(1, _E),
                   W_ih_f, bf.reshape(1, 3 * _H),
                   W_hh_f, b_hh_f.reshape(1, 3 * _H),
                   W_ih_b, bb.reshape(1, 3 * _H),
                   W_hh_b, b_hh_b.reshape(1, 3 * _H),
                   h2l_w, h2l_b.reshape(1, h2l_b.shape[0]))
